# Initial kernel scaffold; baseline (speedup 1.0000x reference)
import jax
import jax.numpy as jnp
from jax import lax
from jax.experimental import pallas as pl
from jax.experimental.pallas import tpu as pltpu
from jax.experimental.pallas import tpu_sc as plsc

N = 50000
E = 1600000
DT = 0.1
NC = 2
NS = 16
NW = NC * NS
EPW = E // NW
CHUNK = 2000
NCHUNKS = EPW // CHUNK
VL = 16
NPAD = 51200
SLC = NPAD // NS


def _seg_body(r_hbm, src_hbm, tgt_hbm, w_hbm, out_hbm,
              r_l, acc, sbuf, tbuf, wbuf, redbuf, outbuf, shared):
    c = lax.axis_index("c")
    s = lax.axis_index("s")
    wid = c * NS + s
    base = wid * EPW

    pltpu.sync_copy(r_hbm, r_l)

    def zbody(i, carry):
        acc[pl.ds(i * VL, VL)] = jnp.zeros((VL,), jnp.float32)
        return carry
    lax.fori_loop(0, NPAD // VL, zbody, 0, unroll=8)

    def cbody(ci, carry):
        off = base + ci * CHUNK
        pltpu.sync_copy(src_hbm.at[pl.ds(off, CHUNK)], sbuf)
        pltpu.sync_copy(tgt_hbm.at[pl.ds(off, CHUNK)], tbuf)
        pltpu.sync_copy(w_hbm.at[pl.ds(off, CHUNK)], wbuf)

        def ibody(j, icarry):
            sl = pl.ds(j * VL, VL)
            sv = sbuf[sl]
            tv = tbuf[sl]
            wv = wbuf[sl]
            rv = plsc.load_gather(r_l, [sv])
            plsc.addupdate_scatter(acc, [tv], rv * wv)
            return icarry
        lax.fori_loop(0, CHUNK // VL, ibody, 0, unroll=4)
        return carry
    lax.fori_loop(0, NCHUNKS, cbody, 0)

    pltpu.sync_copy(acc, shared.at[s])
    plsc.subcore_barrier()

    col = pl.ds(s * SLC, SLC)
    pltpu.sync_copy(shared.at[0, col], outbuf)
    for k in range(1, NS):
        pltpu.sync_copy(shared.at[k, col], redbuf)

        def rbody(i, carry):
            sl2 = pl.ds(i * VL, VL)
            outbuf[sl2] = outbuf[sl2] + redbuf[sl2]
            return carry
        lax.fori_loop(0, SLC // VL, rbody, 0, unroll=8)
    pltpu.sync_copy(outbuf, out_hbm.at[c, col])


_seg = pl.kernel(
    _seg_body,
    out_type=jax.ShapeDtypeStruct((NC, NPAD), jnp.float32),
    mesh=plsc.VectorSubcoreMesh(core_axis_name="c", subcore_axis_name="s"),
    scratch_types=[
        pltpu.VMEM((NPAD,), jnp.float32),
        pltpu.VMEM((NPAD,), jnp.float32),
        pltpu.VMEM((CHUNK,), jnp.int32),
        pltpu.VMEM((CHUNK,), jnp.int32),
        pltpu.VMEM((CHUNK,), jnp.float32),
        pltpu.VMEM((SLC,), jnp.float32),
        pltpu.VMEM((SLC,), jnp.float32),
        pltpu.VMEM_SHARED((NS, NPAD), jnp.float32),
    ],
)


def kernel(tm1_input, weights, tau_params, bias, scale_excitatory,
           scale_inhibitory, source_indices, target_indices, type_ids,
           tm1_positions, steps):
    edge_scales = jnp.where(
        weights > 0, scale_excitatory,
        jnp.where(weights < 0, scale_inhibitory, jnp.ones_like(weights)))
    w = weights * edge_scales
    tau = tau_params[type_ids]
    alpha = DT / tau

    is_tm1 = jnp.zeros((N,), jnp.bool_).at[tm1_positions].set(True)
    stim_full = jnp.zeros((N,), jnp.float32).at[tm1_positions].set(tm1_input[0])

    def step(_, v):
        v_c = jnp.where(is_tm1, stim_full, v)
        r = jax.nn.relu(v_c)
        r_pad = jnp.zeros((NPAD,), jnp.float32).at[:N].set(r)
        parts = _seg(r_pad, source_indices, target_indices, w)
        syn = parts[0, :N] + parts[1, :N]
        v_new = v_c + alpha * (syn + bias - v_c)
        return jnp.where(is_tm1, stim_full, v_new)

    v = lax.fori_loop(0, steps, step, jnp.zeros((N,), jnp.float32))
    return v[None, :]


# speedup vs baseline: 91.6164x; 91.6164x over previous
import jax
import jax.numpy as jnp
from jax import lax
from jax.experimental import pallas as pl
from jax.experimental.pallas import tpu as pltpu
from jax.experimental.pallas import tpu_sc as plsc

N = 50000
E = 1600000
DT = 0.1
NC = 2
NS = 16
NW = NC * NS
CHUNK = 2048
CROWS = CHUNK // 128
NCHUNKS = 25
EPW = CHUNK * NCHUNKS
EPAD = EPW * NW
VL = 16
NPAD = 50176
SLC = NPAD // NS


def _seg_body(r_hbm, src_hbm, tgt_hbm, w_hbm, out_hbm,
              r_l, sbuf, wbuf, tbuf, ebuf, zbuf, acc_s):
    c = lax.axis_index("c")
    s = lax.axis_index("s")
    wid = c * NS + s
    base = wid * EPW

    pltpu.sync_copy(r_hbm, r_l)

    def zb(i, carry):
        zbuf[pl.ds(i * VL, VL)] = jnp.zeros((VL,), jnp.float32)
        return carry
    lax.fori_loop(0, SLC // VL, zb, 0, unroll=8)
    pltpu.sync_copy(zbuf, acc_s.at[pl.ds(s * SLC, SLC)])
    plsc.subcore_barrier()

    def cbody(ci, carry):
        off = base + ci * CHUNK
        pltpu.sync_copy(src_hbm.at[pl.ds(off, CHUNK)], sbuf)
        pltpu.sync_copy(w_hbm.at[pl.ds(off, CHUNK)], wbuf)
        pltpu.sync_copy(tgt_hbm.at[pl.ds(off, CHUNK)], tbuf)

        def ibody(j, icarry):
            sl = pl.ds(j * VL, VL)
            sv = sbuf[sl]
            wv = wbuf[sl]
            rv = plsc.load_gather(r_l, [sv])
            ebuf[sl] = rv * wv
            return icarry
        lax.fori_loop(0, CHUNK // VL, ibody, 0, unroll=4)

        pltpu.sync_copy(ebuf, acc_s.at[tbuf], add=True)
        return carry
    lax.fori_loop(0, NCHUNKS, cbody, 0)
    plsc.subcore_barrier()

    pltpu.sync_copy(acc_s.at[pl.ds(s * SLC, SLC)], zbuf)
    pltpu.sync_copy(zbuf, out_hbm.at[pl.ds(c * NPAD + s * SLC, SLC)])


_seg = pl.kernel(
    _seg_body,
    out_type=jax.ShapeDtypeStruct((NC * NPAD,), jnp.float32),
    mesh=plsc.VectorSubcoreMesh(core_axis_name="c", subcore_axis_name="s"),
    scratch_types=[
        pltpu.VMEM((N,), jnp.float32),
        pltpu.VMEM((CHUNK,), jnp.int32),
        pltpu.VMEM((CHUNK,), jnp.float32),
        pltpu.VMEM((CHUNK,), jnp.int32),
        pltpu.VMEM((CHUNK,), jnp.float32),
        pltpu.VMEM((SLC,), jnp.float32),
        pltpu.VMEM_SHARED((NPAD,), jnp.float32),
    ],
    compiler_params=pltpu.CompilerParams(needs_layout_passes=False),
)


def kernel(tm1_input, weights, tau_params, bias, scale_excitatory,
           scale_inhibitory, source_indices, target_indices, type_ids,
           tm1_positions, steps):
    edge_scales = jnp.where(
        weights > 0, scale_excitatory,
        jnp.where(weights < 0, scale_inhibitory, jnp.ones_like(weights)))
    w = weights * edge_scales

    npadE = EPAD - E
    src_p = jnp.concatenate([source_indices, jnp.zeros((npadE,), jnp.int32)])
    tgt_p = jnp.concatenate(
        [target_indices, jnp.full((npadE,), N, jnp.int32)])
    w_p = jnp.concatenate([w, jnp.zeros((npadE,), jnp.float32)])

    tau = tau_params[type_ids]
    alpha = DT / tau
    is_tm1 = jnp.zeros((N,), jnp.bool_).at[tm1_positions].set(True)
    stim_full = jnp.zeros((N,), jnp.float32).at[tm1_positions].set(tm1_input[0])

    def step(_, v):
        v_c = jnp.where(is_tm1, stim_full, v)
        r = jax.nn.relu(v_c)
        parts = _seg(r, src_p, tgt_p, w_p)
        syn = parts[:N] + parts[NPAD:NPAD + N]
        v_new = v_c + alpha * (syn + bias - v_c)
        return jnp.where(is_tm1, stim_full, v_new)

    v = lax.fori_loop(0, steps, step, jnp.zeros((N,), jnp.float32))
    return v[None, :]
